# baseline (speedup 1.0000x reference)
import functools
import jax
import jax.numpy as jnp
from jax import lax
from jax.experimental import pallas as pl
from jax.experimental.pallas import tpu as pltpu
from jax.experimental.pallas import tpu_sc as plsc

N = 10000
E = 320000
NPAD = 10240
HALF = NPAD // 2
BLK = 1024
NBLK = NPAD // BLK
EPT = E // 16
ECH = 2000
NCH = EPT // ECH
CAP = EPT + 16
RPT = HALF // 16
F32 = jnp.float32


def _feat_body(x_ref, comm_ref, w0_ref, b0_ref, g_ref, beta_ref,
               hf_ref, hsum_ref, cnt_ref):
    i = pl.program_id(0)
    x = x_ref[...]
    hf = jnp.dot(x, w0_ref[...], preferred_element_type=F32)
    hf = (hf + b0_ref[0, 0, :]) * g_ref[0, 0, :] + beta_ref[0, 0, :]
    hf = jnp.maximum(hf, 0.0)
    hf_ref[...] = hf
    c = comm_ref[0, 0, :]
    iot = lax.broadcasted_iota(jnp.int32, (BLK, 128), 1)
    p = (c[:, None] == iot).astype(F32)
    ps = lax.dot_general(p, hf, (((0,), (0,)), ((), ())),
                         preferred_element_type=F32)
    pc = lax.dot_general(p, jnp.ones((BLK, 128), F32),
                         (((0,), (0,)), ((), ())),
                         preferred_element_type=F32)

    @pl.when(i == 0)
    def _():
        hsum_ref[...] = ps
        cnt_ref[...] = pc

    @pl.when(i > 0)
    def _():
        hsum_ref[...] += ps
        cnt_ref[...] += pc


def _feat_pool(x, comm3, w0, b0, g, beta):
    return pl.pallas_call(
        _feat_body,
        grid=(NBLK,),
        in_specs=[
            pl.BlockSpec((BLK, 128), lambda i: (i, 0)),
            pl.BlockSpec((1, 1, BLK), lambda i: (i, 0, 0)),
            pl.BlockSpec((128, 128), lambda i: (0, 0)),
            pl.BlockSpec((1, 1, 128), lambda i: (0, 0, 0)),
            pl.BlockSpec((1, 1, 128), lambda i: (0, 0, 0)),
            pl.BlockSpec((1, 1, 128), lambda i: (0, 0, 0)),
        ],
        out_specs=[
            pl.BlockSpec((BLK, 128), lambda i: (i, 0)),
            pl.BlockSpec((128, 128), lambda i: (0, 0)),
            pl.BlockSpec((128, 128), lambda i: (0, 0)),
        ],
        out_shape=[
            jax.ShapeDtypeStruct((NPAD, 128), F32),
            jax.ShapeDtypeStruct((128, 128), F32),
            jax.ShapeDtypeStruct((128, 128), F32),
        ],
    )(x, comm3, w0, b0, g, beta)


def _mm1_body(hf_ref, comm_ref, hsum_ref, cnt_ref, w1a_ref, w1b_ref,
              as_ref, ad_ref, hwt_ref, aso_ref, ado_ref):
    hf = hf_ref[...]
    c = comm_ref[0, 0, :]
    iot = lax.broadcasted_iota(jnp.int32, (BLK, 128), 1)
    p = (c[:, None] == iot).astype(F32)
    pooled = hsum_ref[...] / jnp.maximum(cnt_ref[...], 1.0)
    q = jnp.dot(pooled, w1b_ref[...], preferred_element_type=F32)
    hw = (jnp.dot(hf, w1a_ref[...], preferred_element_type=F32)
          + jnp.dot(p, q, preferred_element_type=F32))
    hwt_ref[0] = hw
    aso_ref[0, 0, :] = jnp.sum(hw * as_ref[0, 0, :][None, :], axis=1)
    ado_ref[0, 0, :] = jnp.sum(hw * ad_ref[0, 0, :][None, :], axis=1)


def _mm1(hf, comm3, hsum, cnt, w1a, w1b, att_s, att_d):
    return pl.pallas_call(
        _mm1_body,
        grid=(8, NBLK),
        in_specs=[
            pl.BlockSpec((BLK, 128), lambda h, i: (i, 0)),
            pl.BlockSpec((1, 1, BLK), lambda h, i: (i, 0, 0)),
            pl.BlockSpec((128, 128), lambda h, i: (0, 0)),
            pl.BlockSpec((128, 128), lambda h, i: (0, 0)),
            pl.BlockSpec((128, 128), lambda h, i: (0, h)),
            pl.BlockSpec((128, 128), lambda h, i: (0, h)),
            pl.BlockSpec((1, 1, 128), lambda h, i: (h, 0, 0)),
            pl.BlockSpec((1, 1, 128), lambda h, i: (h, 0, 0)),
        ],
        out_specs=[
            pl.BlockSpec((1, BLK, 128), lambda h, i: (h, i, 0)),
            pl.BlockSpec((1, 1, BLK), lambda h, i: (h, 0, i)),
            pl.BlockSpec((1, 1, BLK), lambda h, i: (h, 0, i)),
        ],
        out_shape=[
            jax.ShapeDtypeStruct((8, NPAD, 128), F32),
            jax.ShapeDtypeStruct((8, 1, NPAD), F32),
            jax.ShapeDtypeStruct((8, 1, NPAD), F32),
        ],
    )(hf, comm3, hsum, cnt, w1a, w1b, att_s, att_d)


def _make_edge_kernel(heads):
    mesh = plsc.VectorSubcoreMesh(core_axis_name="c", subcore_axis_name="s")

    @functools.partial(
        pl.kernel,
        mesh=mesh,
        compiler_params=pltpu.CompilerParams(needs_layout_passes=False),
        out_type=(
            jax.ShapeDtypeStruct((2, heads, HALF, 128), F32),
            jax.ShapeDtypeStruct((2, heads, 16, 1, HALF), F32),
        ),
        scratch_types=[
            pltpu.VMEM((N,), jnp.int32),
            pltpu.VMEM((ECH,), jnp.int32),
            pltpu.VMEM((ECH,), jnp.int32),
            pltpu.VMEM((CAP,), jnp.int32),
            pltpu.VMEM((NPAD,), F32),
            pltpu.VMEM((HALF,), F32),
            pltpu.VMEM((16,), F32),
            pltpu.VMEM((64,), jnp.int32),
            pltpu.VMEM((64,), jnp.int32),
            pltpu.VMEM((64, 128), F32),
            pltpu.VMEM((64, 128), F32),
            pltpu.VMEM((64, 128), F32),
            pltpu.VMEM((HALF,), F32),
            pltpu.VMEM_SHARED((HALF, 128), F32),
            pltpu.SemaphoreType.DMA,
        ],
    )
    def k(src_hbm, dst_hbm, comm_hbm, hwt_hbm, as_hbm, ad_hbm, cvec_hbm,
          out_hbm, den_hbm, commv, srcv, dstv, cpk, asv, adv, cvb,
          gidx, sidx, rows, msg, zbuf, denl, acc, sem):
        core = lax.axis_index("c")
        sub = lax.axis_index("s")
        nbase = core * HALF
        ebase = sub * EPT

        pltpu.sync_copy(comm_hbm, commv)
        pltpu.sync_copy(cvec_hbm, cvb)
        cvec = cvb[...]
        iota = lax.broadcasted_iota(jnp.int32, (16,), 0)
        z16 = jnp.zeros((16,), F32)

        def zrow(r, _):
            for cc in range(8):
                zbuf[r, pl.ds(cc * 16, 16)] = z16
            return 0

        lax.fori_loop(0, 64, zrow, 0)

        def cbody(j, off):
            s16 = srcv[pl.ds(j * 16, 16)]
            d16 = dstv[pl.ds(j * 16, 16)]
            cs = plsc.load_gather(commv, [s16])
            cd = plsc.load_gather(commv, [d16])
            dl = d16 - nbase
            m = ((cs == cd) & (dl >= 0) & (dl < HALF))
            pk = (s16 << 13) | (dl & 8191)
            plsc.store_compressed(cpk.at[pl.ds(off, 16)], pk, mask=m)
            npk = jnp.max(plsc.all_reduce_population_count(m))
            return off + npk

        def mcompact(mc, off):
            e0 = pl.multiple_of(ebase + mc * ECH, 8)
            pltpu.sync_copy(src_hbm.at[pl.ds(e0, ECH)], srcv)
            pltpu.sync_copy(dst_hbm.at[pl.ds(e0, ECH)], dstv)
            return lax.fori_loop(0, ECH // 16, cbody, off)

        tot = lax.fori_loop(0, NCH, mcompact, jnp.int32(0))
        nchunk = (tot + 63) // 64

        def dzero(r, _):
            denl[pl.ds(r * 16, 16)] = z16
            return 0

        def hbody(h, _):
            pltpu.sync_copy(as_hbm.at[h].at[0], asv)
            pltpu.sync_copy(ad_hbm.at[h].at[0].at[pl.ds(nbase, HALF)], adv)
            lax.fori_loop(0, HALF // 16, dzero, 0)

            for b in range(RPT // 64):
                pltpu.sync_copy(
                    zbuf, acc.at[pl.ds(sub * RPT + b * 64, 64)])
            plsc.subcore_barrier()

            def ebody(kk, _):
                bs = kk * 64
                evs = []
                for q in range(4):
                    pk = cpk[pl.ds(bs + q * 16, 16)]
                    valid = (bs + q * 16 + iota) < tot
                    pk = jnp.where(valid, pk, 0)
                    s16 = pk >> 13
                    d16 = pk & 8191
                    d16 = jnp.where(valid, d16, 0)
                    a_s = plsc.load_gather(asv, [s16])
                    a_d = plsc.load_gather(adv, [d16])
                    al = a_s + a_d
                    al = jnp.where(al >= 0.0, al, 0.2 * al)
                    ev = jnp.exp(al - cvec)
                    ev = jnp.where(valid, ev, 0.0)
                    gidx[pl.ds(q * 16, 16)] = s16
                    sidx[pl.ds(q * 16, 16)] = d16
                    plsc.addupdate_scatter(denl, [d16], ev)
                    evs.append(ev)
                pltpu.async_copy(hwt_hbm.at[h].at[gidx], rows, sem).wait()
                for q in range(4):
                    for j in range(16):
                        ej = jnp.sum(jnp.where(iota == j, evs[q], 0.0))
                        r = q * 16 + j
                        for cc in range(8):
                            msg[r, pl.ds(cc * 16, 16)] = (
                                rows[r, pl.ds(cc * 16, 16)] * ej)
                pltpu.sync_copy(msg, acc.at[sidx], add=True)
                return 0

            lax.fori_loop(0, nchunk, ebody, 0)
            plsc.subcore_barrier()

            for b in range(RPT // 64):
                r0 = sub * RPT + b * 64
                pltpu.sync_copy(
                    acc.at[pl.ds(r0, 64)],
                    out_hbm.at[core].at[h].at[pl.ds(r0, 64)])
            pltpu.sync_copy(denl, den_hbm.at[core].at[h].at[sub].at[0])
            plsc.subcore_barrier()
            return 0

        lax.fori_loop(0, heads, hbody, 0)

    return k


def _densum_body(den_ref, out_ref):
    t = pl.program_id(1)
    d = den_ref[...][:, :, 0, 0, :]

    @pl.when(t == 0)
    def _():
        out_ref[...] = d

    @pl.when(t > 0)
    def _():
        out_ref[...] += d


def _densum(denp, hh):
    denp5 = denp
    return pl.pallas_call(
        _densum_body,
        grid=(2, 16),
        in_specs=[pl.BlockSpec((1, hh, 1, 1, HALF),
                               lambda c, t: (c, 0, t, 0, 0))],
        out_specs=pl.BlockSpec((1, hh, HALF), lambda c, t: (c, 0, 0)),
        out_shape=jax.ShapeDtypeStruct((2, hh, HALF), F32),
    )(denp5)


def _mm2_body(acc_ref, den_ref, b1_ref, w2_ref, as2_ref, ad2_ref,
              hw2_ref, aso_ref, ado_ref):
    a = acc_ref[0]
    den = den_ref[0]
    out = jnp.zeros((BLK, 128), F32)
    for h in range(8):
        num = a[h]
        dh = den[h][:, None]
        o = jnp.where(dh > 0.0, num / jnp.where(dh > 0.0, dh, 1.0), 0.0)
        o = o + b1_ref[0, h, :][None, :]
        o = jnp.where(o > 0.0, o, jnp.exp(jnp.minimum(o, 0.0)) - 1.0)
        out = out + jnp.dot(o, w2_ref[h], preferred_element_type=F32)
    hw2_ref[...] = out
    aso_ref[0, 0, :] = jnp.sum(out * as2_ref[0, 0, :][None, :], axis=1)
    ado_ref[0, 0, :] = jnp.sum(out * ad2_ref[0, 0, :][None, :], axis=1)


def _mm2(acc1, den1, b1m, w2m, as2, ad2):
    return pl.pallas_call(
        _mm2_body,
        grid=(NBLK,),
        in_specs=[
            pl.BlockSpec((1, 8, BLK, 128),
                         lambda i: (i // 5, 0, i % 5, 0)),
            pl.BlockSpec((1, 8, BLK), lambda i: (i // 5, 0, i % 5)),
            pl.BlockSpec((1, 8, 128), lambda i: (0, 0, 0)),
            pl.BlockSpec((8, 128, 128), lambda i: (0, 0, 0)),
            pl.BlockSpec((1, 1, 128), lambda i: (0, 0, 0)),
            pl.BlockSpec((1, 1, 128), lambda i: (0, 0, 0)),
        ],
        out_specs=[
            pl.BlockSpec((BLK, 128), lambda i: (i, 0)),
            pl.BlockSpec((1, 1, BLK), lambda i: (0, 0, i)),
            pl.BlockSpec((1, 1, BLK), lambda i: (0, 0, i)),
        ],
        out_shape=[
            jax.ShapeDtypeStruct((NPAD, 128), F32),
            jax.ShapeDtypeStruct((1, 1, NPAD), F32),
            jax.ShapeDtypeStruct((1, 1, NPAD), F32),
        ],
    )(acc1, den1, b1m, w2m, as2, ad2)


def _fin_body(acc_ref, den_ref, b2_ref, wp1_ref, bp1_ref, wp2_ref, bp2_ref,
              zn_ref, zp_ref):
    a = acc_ref[0, 0]
    den = den_ref[0, 0][:, None]
    z = jnp.where(den > 0.0, a / jnp.where(den > 0.0, den, 1.0), 0.0)
    z = z + b2_ref[0, 0, :][None, :]
    zn_ref[...] = z
    t = jnp.dot(z, wp1_ref[...], preferred_element_type=F32)
    t = jnp.maximum(t + bp1_ref[0, 0, :][None, :], 0.0)
    t = jnp.dot(t, wp2_ref[...], preferred_element_type=F32)
    zp_ref[...] = t + bp2_ref[0, 0, :][None, :]


def _final(acc2, den2, b2m, wp1, bp1m, wp2, bp2m):
    return pl.pallas_call(
        _fin_body,
        grid=(NBLK,),
        in_specs=[
            pl.BlockSpec((1, 1, BLK, 128),
                         lambda i: (i // 5, 0, i % 5, 0)),
            pl.BlockSpec((1, 1, BLK), lambda i: (i // 5, 0, i % 5)),
            pl.BlockSpec((1, 1, 128), lambda i: (0, 0, 0)),
            pl.BlockSpec((128, 128), lambda i: (0, 0)),
            pl.BlockSpec((1, 1, 128), lambda i: (0, 0, 0)),
            pl.BlockSpec((128, 128), lambda i: (0, 0)),
            pl.BlockSpec((1, 1, 128), lambda i: (0, 0, 0)),
        ],
        out_specs=[
            pl.BlockSpec((BLK, 128), lambda i: (i, 0)),
            pl.BlockSpec((BLK, 128), lambda i: (i, 0)),
        ],
        out_shape=[
            jax.ShapeDtypeStruct((NPAD, 128), F32),
            jax.ShapeDtypeStruct((NPAD, 128), F32),
        ],
    )(acc2, den2, b2m, wp1, bp1m, wp2, bp2m)


_edge8 = _make_edge_kernel(8)
_edge1 = _make_edge_kernel(1)


def kernel(x, edge_index, community_ids, W0, b0, bn_gamma, bn_beta, W1,
           att_src1, att_dst1, b1, W2, att_src2, att_dst2, b2, Wp1, bp1,
           Wp2, bp2):
    xp = jnp.pad(x, ((0, NPAD - N), (0, 0)))
    commp = jnp.pad(community_ids, (0, NPAD - N), constant_values=127)
    comm3 = commp.reshape(NBLK, 1, BLK)
    g = (bn_gamma / jnp.sqrt(1.0 + 1e-5)).reshape(1, 1, 128)
    b0m = b0.reshape(1, 1, 128)
    betam = bn_beta.reshape(1, 1, 128)

    hf, hsum, cnt = _feat_pool(xp, comm3, W0, b0m, g, betam)

    w1a = W1[:128]
    w1b = W1[128:]
    as1 = att_src1.reshape(8, 1, 128)
    ad1 = att_dst1.reshape(8, 1, 128)
    hwt, a_s, a_d = _mm1(hf, comm3, hsum, cnt, w1a, w1b, as1, ad1)

    src = edge_index[0]
    dst = edge_index[1]
    c1 = jnp.max(a_s) + jnp.max(a_d)
    c1 = jnp.where(c1 >= 0.0, c1, 0.2 * c1)
    cvec1 = jnp.full((16,), c1, F32)
    acc1, den1 = _edge8(src, dst, community_ids, hwt, a_s, a_d, cvec1)
    den1r = _densum(den1, 8)

    b1m = b1.reshape(1, 8, 128)
    w2m = W2.reshape(8, 128, 128)
    as2m = att_src2.reshape(1, 1, 128)
    ad2m = att_dst2.reshape(1, 1, 128)
    hw2, as2o, ad2o = _mm2(acc1, den1r, b1m, w2m, as2m, ad2m)

    c2 = jnp.max(as2o) + jnp.max(ad2o)
    c2 = jnp.where(c2 >= 0.0, c2, 0.2 * c2)
    cvec2 = jnp.full((16,), c2, F32)
    hw2t = hw2.reshape(1, NPAD, 128)
    acc2, den2 = _edge1(src, dst, community_ids, hw2t, as2o, ad2o, cvec2)
    den2r = _densum(den2, 1)

    b2m = b2.reshape(1, 1, 128)
    bp1m = bp1.reshape(1, 1, 128)
    bp2m = bp2.reshape(1, 1, 128)
    zn, zp = _final(acc2, den2r, b2m, Wp1, bp1m, Wp2, bp2m)
    return (zn[:N], zp[:N])


# speedup vs baseline: 1.3371x; 1.3371x over previous
import functools
import jax
import jax.numpy as jnp
from jax import lax
from jax.experimental import pallas as pl
from jax.experimental.pallas import tpu as pltpu
from jax.experimental.pallas import tpu_sc as plsc

N = 10000
E = 320000
NPAD = 10240
HALF = NPAD // 2
BLK = 1024
NBLK = NPAD // BLK
EPT = E // 16
ECH = 2000
NCH = EPT // ECH
CAP = EPT + 16
RPT = HALF // 16
F32 = jnp.float32


def _feat_body(x_ref, comm_ref, w0_ref, b0_ref, g_ref, beta_ref,
               hf_ref, hsum_ref, cnt_ref):
    i = pl.program_id(0)
    x = x_ref[...]
    hf = jnp.dot(x, w0_ref[...], preferred_element_type=F32)
    hf = (hf + b0_ref[0, 0, :]) * g_ref[0, 0, :] + beta_ref[0, 0, :]
    hf = jnp.maximum(hf, 0.0)
    hf_ref[...] = hf
    c = comm_ref[0, 0, :]
    iot = lax.broadcasted_iota(jnp.int32, (BLK, 128), 1)
    p = (c[:, None] == iot).astype(F32)
    ps = lax.dot_general(p, hf, (((0,), (0,)), ((), ())),
                         preferred_element_type=F32)
    pc = lax.dot_general(p, jnp.ones((BLK, 128), F32),
                         (((0,), (0,)), ((), ())),
                         preferred_element_type=F32)

    @pl.when(i == 0)
    def _():
        hsum_ref[...] = ps
        cnt_ref[...] = pc

    @pl.when(i > 0)
    def _():
        hsum_ref[...] += ps
        cnt_ref[...] += pc


def _feat_pool(x, comm3, w0, b0, g, beta):
    return pl.pallas_call(
        _feat_body,
        grid=(NBLK,),
        in_specs=[
            pl.BlockSpec((BLK, 128), lambda i: (i, 0)),
            pl.BlockSpec((1, 1, BLK), lambda i: (i, 0, 0)),
            pl.BlockSpec((128, 128), lambda i: (0, 0)),
            pl.BlockSpec((1, 1, 128), lambda i: (0, 0, 0)),
            pl.BlockSpec((1, 1, 128), lambda i: (0, 0, 0)),
            pl.BlockSpec((1, 1, 128), lambda i: (0, 0, 0)),
        ],
        out_specs=[
            pl.BlockSpec((BLK, 128), lambda i: (i, 0)),
            pl.BlockSpec((128, 128), lambda i: (0, 0)),
            pl.BlockSpec((128, 128), lambda i: (0, 0)),
        ],
        out_shape=[
            jax.ShapeDtypeStruct((NPAD, 128), F32),
            jax.ShapeDtypeStruct((128, 128), F32),
            jax.ShapeDtypeStruct((128, 128), F32),
        ],
    )(x, comm3, w0, b0, g, beta)


def _mm1_body(hf_ref, comm_ref, hsum_ref, cnt_ref, w1a_ref, w1b_ref,
              as_ref, ad_ref, hwt_ref, aso_ref, ado_ref):
    hf = hf_ref[...]
    c = comm_ref[0, 0, :]
    iot = lax.broadcasted_iota(jnp.int32, (BLK, 128), 1)
    p = (c[:, None] == iot).astype(F32)
    pooled = hsum_ref[...] / jnp.maximum(cnt_ref[...], 1.0)
    q = jnp.dot(pooled, w1b_ref[...], preferred_element_type=F32)
    hw = (jnp.dot(hf, w1a_ref[...], preferred_element_type=F32)
          + jnp.dot(p, q, preferred_element_type=F32))
    hwt_ref[0] = hw
    aso_ref[0, 0, :] = jnp.sum(hw * as_ref[0, 0, :][None, :], axis=1)
    ado_ref[0, 0, :] = jnp.sum(hw * ad_ref[0, 0, :][None, :], axis=1)


def _mm1(hf, comm3, hsum, cnt, w1a, w1b, att_s, att_d):
    return pl.pallas_call(
        _mm1_body,
        grid=(8, NBLK),
        in_specs=[
            pl.BlockSpec((BLK, 128), lambda h, i: (i, 0)),
            pl.BlockSpec((1, 1, BLK), lambda h, i: (i, 0, 0)),
            pl.BlockSpec((128, 128), lambda h, i: (0, 0)),
            pl.BlockSpec((128, 128), lambda h, i: (0, 0)),
            pl.BlockSpec((128, 128), lambda h, i: (0, h)),
            pl.BlockSpec((128, 128), lambda h, i: (0, h)),
            pl.BlockSpec((1, 1, 128), lambda h, i: (h, 0, 0)),
            pl.BlockSpec((1, 1, 128), lambda h, i: (h, 0, 0)),
        ],
        out_specs=[
            pl.BlockSpec((1, BLK, 128), lambda h, i: (h, i, 0)),
            pl.BlockSpec((1, 1, BLK), lambda h, i: (h, 0, i)),
            pl.BlockSpec((1, 1, BLK), lambda h, i: (h, 0, i)),
        ],
        out_shape=[
            jax.ShapeDtypeStruct((8, NPAD, 128), F32),
            jax.ShapeDtypeStruct((8, 1, NPAD), F32),
            jax.ShapeDtypeStruct((8, 1, NPAD), F32),
        ],
    )(hf, comm3, hsum, cnt, w1a, w1b, att_s, att_d)


def _make_edge_kernel(heads):
    mesh = plsc.VectorSubcoreMesh(core_axis_name="c", subcore_axis_name="s")

    @functools.partial(
        pl.kernel,
        mesh=mesh,
        compiler_params=pltpu.CompilerParams(needs_layout_passes=False),
        out_type=(
            jax.ShapeDtypeStruct((2, heads, HALF, 128), F32),
            jax.ShapeDtypeStruct((2, heads, 16, 1, HALF), F32),
        ),
        scratch_types=[
            pltpu.VMEM((N,), jnp.int32),
            pltpu.VMEM((ECH,), jnp.int32),
            pltpu.VMEM((ECH,), jnp.int32),
            pltpu.VMEM((CAP,), jnp.int32),
            pltpu.VMEM((CAP,), jnp.int32),
            pltpu.VMEM((NPAD,), F32),
            pltpu.VMEM((HALF,), F32),
            pltpu.VMEM((16,), F32),
            pltpu.VMEM((16,), jnp.int32),
            pltpu.VMEM((16,), jnp.int32),
            pltpu.VMEM((16, 128), F32),
            pltpu.VMEM((16, 128), F32),
            pltpu.VMEM((64, 128), F32),
            pltpu.VMEM((HALF,), F32),
            pltpu.VMEM_SHARED((HALF, 128), F32),
            pltpu.SemaphoreType.DMA,
        ],
    )
    def k(src_hbm, dst_hbm, comm_hbm, hwt_hbm, as_hbm, ad_hbm, cvec_hbm,
          out_hbm, den_hbm, commv, srcv, dstv, csrc, cdst, asv, adv, cvb,
          gidx, sidx, rows, msg, zbuf, denl, acc, sem):
        core = lax.axis_index("c")
        sub = lax.axis_index("s")
        nbase = core * HALF
        ebase = sub * EPT

        pltpu.sync_copy(comm_hbm, commv)
        pltpu.sync_copy(cvec_hbm, cvb)
        cvec = cvb[...]
        iota = lax.broadcasted_iota(jnp.int32, (16,), 0)
        z16 = jnp.zeros((16,), F32)

        def zrow(r, _):
            for cc in range(8):
                zbuf[r, pl.ds(cc * 16, 16)] = z16
            return 0

        lax.fori_loop(0, 64, zrow, 0)

        def cbody(j, off):
            s16 = srcv[pl.ds(j * 16, 16)]
            d16 = dstv[pl.ds(j * 16, 16)]
            cs = plsc.load_gather(commv, [s16])
            cd = plsc.load_gather(commv, [d16])
            dl = d16 - nbase
            m = ((cs == cd) & (dl >= 0) & (dl < HALF))
            plsc.store_compressed(csrc.at[pl.ds(off, 16)], s16, mask=m)
            plsc.store_compressed(cdst.at[pl.ds(off, 16)], dl, mask=m)
            npk = jnp.max(plsc.all_reduce_population_count(m))
            return off + npk

        def mcompact(mc, off):
            e0 = pl.multiple_of(ebase + mc * ECH, 8)
            pltpu.sync_copy(src_hbm.at[pl.ds(e0, ECH)], srcv)
            pltpu.sync_copy(dst_hbm.at[pl.ds(e0, ECH)], dstv)
            return lax.fori_loop(0, ECH // 16, cbody, off)

        tot = lax.fori_loop(0, NCH, mcompact, jnp.int32(0))
        nchunk = (tot + 15) // 16

        def dzero(r, _):
            denl[pl.ds(r * 16, 16)] = z16
            return 0

        def hbody(h, _):
            pltpu.sync_copy(as_hbm.at[h].at[0], asv)
            pltpu.sync_copy(ad_hbm.at[h].at[0].at[pl.ds(nbase, HALF)], adv)
            lax.fori_loop(0, HALF // 16, dzero, 0)

            for b in range(RPT // 64):
                pltpu.sync_copy(
                    zbuf, acc.at[pl.ds(sub * RPT + b * 64, 64)])
            plsc.subcore_barrier()

            def ebody(kk, _):
                bs = kk * 16
                s16 = csrc[pl.ds(bs, 16)]
                d16 = cdst[pl.ds(bs, 16)]
                valid = (bs + iota) < tot
                s16 = jnp.where(valid, s16, 0)
                d16 = jnp.where(valid, d16, 0)
                a_s = plsc.load_gather(asv, [s16])
                a_d = plsc.load_gather(adv, [d16])
                al = a_s + a_d
                al = jnp.where(al >= 0.0, al, 0.2 * al)
                ev = jnp.exp(al - cvec)
                ev = jnp.where(valid, ev, 0.0)
                gidx[...] = s16
                sidx[...] = d16
                pltpu.async_copy(hwt_hbm.at[h].at[gidx], rows, sem).wait()
                plsc.addupdate_scatter(denl, [d16], ev)
                for j in range(16):
                    ej = jnp.sum(jnp.where(iota == j, ev, 0.0))
                    for cc in range(8):
                        msg[j, pl.ds(cc * 16, 16)] = (
                            rows[j, pl.ds(cc * 16, 16)] * ej)
                pltpu.sync_copy(msg, acc.at[sidx], add=True)
                return 0

            lax.fori_loop(0, nchunk, ebody, 0)
            plsc.subcore_barrier()

            for b in range(RPT // 64):
                r0 = sub * RPT + b * 64
                pltpu.sync_copy(
                    acc.at[pl.ds(r0, 64)],
                    out_hbm.at[core].at[h].at[pl.ds(r0, 64)])
            pltpu.sync_copy(denl, den_hbm.at[core].at[h].at[sub].at[0])
            plsc.subcore_barrier()
            return 0

        lax.fori_loop(0, heads, hbody, 0)

    return k


def _densum_body(den_ref, out_ref):
    t = pl.program_id(1)
    d = den_ref[...][:, :, 0, 0, :]

    @pl.when(t == 0)
    def _():
        out_ref[...] = d

    @pl.when(t > 0)
    def _():
        out_ref[...] += d


def _densum(denp, hh):
    denp5 = denp
    return pl.pallas_call(
        _densum_body,
        grid=(2, 16),
        in_specs=[pl.BlockSpec((1, hh, 1, 1, HALF),
                               lambda c, t: (c, 0, t, 0, 0))],
        out_specs=pl.BlockSpec((1, hh, HALF), lambda c, t: (c, 0, 0)),
        out_shape=jax.ShapeDtypeStruct((2, hh, HALF), F32),
    )(denp5)


def _mm2_body(acc_ref, den_ref, b1_ref, w2_ref, as2_ref, ad2_ref,
              hw2_ref, aso_ref, ado_ref):
    a = acc_ref[0]
    den = den_ref[0]
    out = jnp.zeros((BLK, 128), F32)
    for h in range(8):
        num = a[h]
        dh = den[h][:, None]
        o = jnp.where(dh > 0.0, num / jnp.where(dh > 0.0, dh, 1.0), 0.0)
        o = o + b1_ref[0, h, :][None, :]
        o = jnp.where(o > 0.0, o, jnp.exp(jnp.minimum(o, 0.0)) - 1.0)
        out = out + jnp.dot(o, w2_ref[h], preferred_element_type=F32)
    hw2_ref[...] = out
    aso_ref[0, 0, :] = jnp.sum(out * as2_ref[0, 0, :][None, :], axis=1)
    ado_ref[0, 0, :] = jnp.sum(out * ad2_ref[0, 0, :][None, :], axis=1)


def _mm2(acc1, den1, b1m, w2m, as2, ad2):
    return pl.pallas_call(
        _mm2_body,
        grid=(NBLK,),
        in_specs=[
            pl.BlockSpec((1, 8, BLK, 128),
                         lambda i: (i // 5, 0, i % 5, 0)),
            pl.BlockSpec((1, 8, BLK), lambda i: (i // 5, 0, i % 5)),
            pl.BlockSpec((1, 8, 128), lambda i: (0, 0, 0)),
            pl.BlockSpec((8, 128, 128), lambda i: (0, 0, 0)),
            pl.BlockSpec((1, 1, 128), lambda i: (0, 0, 0)),
            pl.BlockSpec((1, 1, 128), lambda i: (0, 0, 0)),
        ],
        out_specs=[
            pl.BlockSpec((BLK, 128), lambda i: (i, 0)),
            pl.BlockSpec((1, 1, BLK), lambda i: (0, 0, i)),
            pl.BlockSpec((1, 1, BLK), lambda i: (0, 0, i)),
        ],
        out_shape=[
            jax.ShapeDtypeStruct((NPAD, 128), F32),
            jax.ShapeDtypeStruct((1, 1, NPAD), F32),
            jax.ShapeDtypeStruct((1, 1, NPAD), F32),
        ],
    )(acc1, den1, b1m, w2m, as2, ad2)


def _fin_body(acc_ref, den_ref, b2_ref, wp1_ref, bp1_ref, wp2_ref, bp2_ref,
              zn_ref, zp_ref):
    a = acc_ref[0, 0]
    den = den_ref[0, 0][:, None]
    z = jnp.where(den > 0.0, a / jnp.where(den > 0.0, den, 1.0), 0.0)
    z = z + b2_ref[0, 0, :][None, :]
    zn_ref[...] = z
    t = jnp.dot(z, wp1_ref[...], preferred_element_type=F32)
    t = jnp.maximum(t + bp1_ref[0, 0, :][None, :], 0.0)
    t = jnp.dot(t, wp2_ref[...], preferred_element_type=F32)
    zp_ref[...] = t + bp2_ref[0, 0, :][None, :]


def _final(acc2, den2, b2m, wp1, bp1m, wp2, bp2m):
    return pl.pallas_call(
        _fin_body,
        grid=(NBLK,),
        in_specs=[
            pl.BlockSpec((1, 1, BLK, 128),
                         lambda i: (i // 5, 0, i % 5, 0)),
            pl.BlockSpec((1, 1, BLK), lambda i: (i // 5, 0, i % 5)),
            pl.BlockSpec((1, 1, 128), lambda i: (0, 0, 0)),
            pl.BlockSpec((128, 128), lambda i: (0, 0)),
            pl.BlockSpec((1, 1, 128), lambda i: (0, 0, 0)),
            pl.BlockSpec((128, 128), lambda i: (0, 0)),
            pl.BlockSpec((1, 1, 128), lambda i: (0, 0, 0)),
        ],
        out_specs=[
            pl.BlockSpec((BLK, 128), lambda i: (i, 0)),
            pl.BlockSpec((BLK, 128), lambda i: (i, 0)),
        ],
        out_shape=[
            jax.ShapeDtypeStruct((NPAD, 128), F32),
            jax.ShapeDtypeStruct((NPAD, 128), F32),
        ],
    )(acc2, den2, b2m, wp1, bp1m, wp2, bp2m)


_edge8 = _make_edge_kernel(8)
_edge1 = _make_edge_kernel(1)


def kernel(x, edge_index, community_ids, W0, b0, bn_gamma, bn_beta, W1,
           att_src1, att_dst1, b1, W2, att_src2, att_dst2, b2, Wp1, bp1,
           Wp2, bp2):
    xp = jnp.pad(x, ((0, NPAD - N), (0, 0)))
    commp = jnp.pad(community_ids, (0, NPAD - N), constant_values=127)
    comm3 = commp.reshape(NBLK, 1, BLK)
    g = (bn_gamma / jnp.sqrt(1.0 + 1e-5)).reshape(1, 1, 128)
    b0m = b0.reshape(1, 1, 128)
    betam = bn_beta.reshape(1, 1, 128)

    hf, hsum, cnt = _feat_pool(xp, comm3, W0, b0m, g, betam)

    w1a = W1[:128]
    w1b = W1[128:]
    as1 = att_src1.reshape(8, 1, 128)
    ad1 = att_dst1.reshape(8, 1, 128)
    hwt, a_s, a_d = _mm1(hf, comm3, hsum, cnt, w1a, w1b, as1, ad1)

    src = edge_index[0]
    dst = edge_index[1]
    c1 = jnp.max(a_s) + jnp.max(a_d)
    c1 = jnp.where(c1 >= 0.0, c1, 0.2 * c1)
    cvec1 = jnp.full((16,), c1, F32)
    acc1, den1 = _edge8(src, dst, community_ids, hwt, a_s, a_d, cvec1)
    den1r = _densum(den1, 8)

    b1m = b1.reshape(1, 8, 128)
    w2m = W2.reshape(8, 128, 128)
    as2m = att_src2.reshape(1, 1, 128)
    ad2m = att_dst2.reshape(1, 1, 128)
    hw2, as2o, ad2o = _mm2(acc1, den1r, b1m, w2m, as2m, ad2m)

    c2 = jnp.max(as2o) + jnp.max(ad2o)
    c2 = jnp.where(c2 >= 0.0, c2, 0.2 * c2)
    cvec2 = jnp.full((16,), c2, F32)
    hw2t = hw2.reshape(1, NPAD, 128)
    acc2, den2 = _edge1(src, dst, community_ids, hw2t, as2o, ad2o, cvec2)
    den2r = _densum(den2, 1)

    b2m = b2.reshape(1, 1, 128)
    bp1m = bp1.reshape(1, 1, 128)
    bp2m = bp2.reshape(1, 1, 128)
    zn, zp = _final(acc2, den2r, b2m, Wp1, bp1m, Wp2, bp2m)
    return (zn[:N], zp[:N])


# speedup vs baseline: 1.3402x; 1.0023x over previous
import functools
import jax
import jax.numpy as jnp
from jax import lax
from jax.experimental import pallas as pl
from jax.experimental.pallas import tpu as pltpu
from jax.experimental.pallas import tpu_sc as plsc

N = 10000
E = 320000
NPAD = 10240
HALF = NPAD // 2
BLK = 1024
NBLK = NPAD // BLK
EPT = E // 16
ECH = 2000
NCH = EPT // ECH
CAP = EPT + 16
RPT = HALF // 16
F32 = jnp.float32


def _feat_body(x_ref, comm_ref, w0_ref, b0_ref, g_ref, beta_ref,
               hf_ref, hsum_ref, cnt_ref):
    i = pl.program_id(0)
    x = x_ref[...]
    hf = jnp.dot(x, w0_ref[...], preferred_element_type=F32)
    hf = (hf + b0_ref[0, 0, :]) * g_ref[0, 0, :] + beta_ref[0, 0, :]
    hf = jnp.maximum(hf, 0.0)
    hf_ref[...] = hf
    c = comm_ref[0, 0, :]
    iot = lax.broadcasted_iota(jnp.int32, (BLK, 128), 1)
    p = (c[:, None] == iot).astype(F32)
    ps = lax.dot_general(p, hf, (((0,), (0,)), ((), ())),
                         preferred_element_type=F32)
    pc = lax.dot_general(p, jnp.ones((BLK, 128), F32),
                         (((0,), (0,)), ((), ())),
                         preferred_element_type=F32)

    @pl.when(i == 0)
    def _():
        hsum_ref[...] = ps
        cnt_ref[...] = pc

    @pl.when(i > 0)
    def _():
        hsum_ref[...] += ps
        cnt_ref[...] += pc


def _feat_pool(x, comm3, w0, b0, g, beta):
    return pl.pallas_call(
        _feat_body,
        grid=(NBLK,),
        in_specs=[
            pl.BlockSpec((BLK, 128), lambda i: (i, 0)),
            pl.BlockSpec((1, 1, BLK), lambda i: (i, 0, 0)),
            pl.BlockSpec((128, 128), lambda i: (0, 0)),
            pl.BlockSpec((1, 1, 128), lambda i: (0, 0, 0)),
            pl.BlockSpec((1, 1, 128), lambda i: (0, 0, 0)),
            pl.BlockSpec((1, 1, 128), lambda i: (0, 0, 0)),
        ],
        out_specs=[
            pl.BlockSpec((BLK, 128), lambda i: (i, 0)),
            pl.BlockSpec((128, 128), lambda i: (0, 0)),
            pl.BlockSpec((128, 128), lambda i: (0, 0)),
        ],
        out_shape=[
            jax.ShapeDtypeStruct((NPAD, 128), F32),
            jax.ShapeDtypeStruct((128, 128), F32),
            jax.ShapeDtypeStruct((128, 128), F32),
        ],
    )(x, comm3, w0, b0, g, beta)


def _mm1_body(hf_ref, comm_ref, hsum_ref, cnt_ref, w1a_ref, w1b_ref,
              as_ref, ad_ref, hwt_ref, aso_ref, ado_ref):
    hf = hf_ref[...]
    c = comm_ref[0, 0, :]
    iot = lax.broadcasted_iota(jnp.int32, (BLK, 128), 1)
    p = (c[:, None] == iot).astype(F32)
    pooled = hsum_ref[...] / jnp.maximum(cnt_ref[...], 1.0)
    q = jnp.dot(pooled, w1b_ref[...], preferred_element_type=F32)
    hw = (jnp.dot(hf, w1a_ref[...], preferred_element_type=F32)
          + jnp.dot(p, q, preferred_element_type=F32))
    hwt_ref[0] = hw
    aso_ref[0, 0, :] = jnp.sum(hw * as_ref[0, 0, :][None, :], axis=1)
    ado_ref[0, 0, :] = jnp.sum(hw * ad_ref[0, 0, :][None, :], axis=1)


def _mm1(hf, comm3, hsum, cnt, w1a, w1b, att_s, att_d):
    return pl.pallas_call(
        _mm1_body,
        grid=(8, NBLK),
        in_specs=[
            pl.BlockSpec((BLK, 128), lambda h, i: (i, 0)),
            pl.BlockSpec((1, 1, BLK), lambda h, i: (i, 0, 0)),
            pl.BlockSpec((128, 128), lambda h, i: (0, 0)),
            pl.BlockSpec((128, 128), lambda h, i: (0, 0)),
            pl.BlockSpec((128, 128), lambda h, i: (0, h)),
            pl.BlockSpec((128, 128), lambda h, i: (0, h)),
            pl.BlockSpec((1, 1, 128), lambda h, i: (h, 0, 0)),
            pl.BlockSpec((1, 1, 128), lambda h, i: (h, 0, 0)),
        ],
        out_specs=[
            pl.BlockSpec((1, BLK, 128), lambda h, i: (h, i, 0)),
            pl.BlockSpec((1, 1, BLK), lambda h, i: (h, 0, i)),
            pl.BlockSpec((1, 1, BLK), lambda h, i: (h, 0, i)),
        ],
        out_shape=[
            jax.ShapeDtypeStruct((8, NPAD, 128), F32),
            jax.ShapeDtypeStruct((8, 1, NPAD), F32),
            jax.ShapeDtypeStruct((8, 1, NPAD), F32),
        ],
    )(hf, comm3, hsum, cnt, w1a, w1b, att_s, att_d)


def _make_edge_kernel(heads):
    mesh = plsc.VectorSubcoreMesh(core_axis_name="c", subcore_axis_name="s")

    @functools.partial(
        pl.kernel,
        mesh=mesh,
        compiler_params=pltpu.CompilerParams(needs_layout_passes=False),
        out_type=(
            jax.ShapeDtypeStruct((2, heads, HALF, 128), F32),
            jax.ShapeDtypeStruct((2, heads, 16, 1, HALF), F32),
        ),
        scratch_types=[
            pltpu.VMEM((N,), jnp.int32),
            pltpu.VMEM((ECH,), jnp.int32),
            pltpu.VMEM((ECH,), jnp.int32),
            pltpu.VMEM((CAP,), jnp.int32),
            pltpu.VMEM((CAP,), jnp.int32),
            pltpu.VMEM((NPAD,), F32),
            pltpu.VMEM((HALF,), F32),
            pltpu.VMEM((16,), F32),
            pltpu.VMEM((16,), jnp.int32),
            pltpu.VMEM((16,), jnp.int32),
            pltpu.VMEM((16, 128), F32),
            pltpu.VMEM((16, 128), F32),
            pltpu.VMEM((HALF,), F32),
            pltpu.VMEM_SHARED((HALF, 128), F32),
            pltpu.SemaphoreType.DMA,
        ],
    )
    def k(src_hbm, dst_hbm, comm_hbm, hwt_hbm, as_hbm, ad_hbm, cvec_hbm,
          za_hbm, zd_hbm, out_hbm, den_hbm, commv, srcv, dstv, csrc, cdst,
          asv, adv, cvb, gidx, sidx, rows, msg, denl, acc, sem):
        core = lax.axis_index("c")
        sub = lax.axis_index("s")
        nbase = core * HALF
        ebase = sub * EPT

        pltpu.sync_copy(comm_hbm, commv)
        pltpu.sync_copy(cvec_hbm, cvb)
        cvec = cvb[...]
        iota = lax.broadcasted_iota(jnp.int32, (16,), 0)
        z16 = jnp.zeros((16,), F32)

        def cbody(j, off):
            s16 = srcv[pl.ds(j * 16, 16)]
            d16 = dstv[pl.ds(j * 16, 16)]
            cs = plsc.load_gather(commv, [s16])
            cd = plsc.load_gather(commv, [d16])
            dl = d16 - nbase
            m = ((cs == cd) & (dl >= 0) & (dl < HALF))
            plsc.store_compressed(csrc.at[pl.ds(off, 16)], s16, mask=m)
            plsc.store_compressed(cdst.at[pl.ds(off, 16)], dl, mask=m)
            npk = jnp.max(plsc.all_reduce_population_count(m))
            return off + npk

        def mcompact(mc, off):
            e0 = pl.multiple_of(ebase + mc * ECH, 8)
            pltpu.sync_copy(src_hbm.at[pl.ds(e0, ECH)], srcv)
            pltpu.sync_copy(dst_hbm.at[pl.ds(e0, ECH)], dstv)
            return lax.fori_loop(0, ECH // 16, cbody, off)

        tot = lax.fori_loop(0, NCH, mcompact, jnp.int32(0))
        nchunk = (tot + 15) // 16

        def hbody(h, _):
            pltpu.sync_copy(as_hbm.at[h].at[0], asv)
            pltpu.sync_copy(ad_hbm.at[h].at[0].at[pl.ds(nbase, HALF)], adv)
            pltpu.sync_copy(zd_hbm, denl)
            pltpu.sync_copy(za_hbm, acc.at[pl.ds(sub * RPT, RPT)])
            plsc.subcore_barrier()

            def ebody(kk, _):
                bs = kk * 16
                s16 = csrc[pl.ds(bs, 16)]
                d16 = cdst[pl.ds(bs, 16)]
                valid = (bs + iota) < tot
                s16 = jnp.where(valid, s16, 0)
                d16 = jnp.where(valid, d16, 0)
                a_s = plsc.load_gather(asv, [s16])
                a_d = plsc.load_gather(adv, [d16])
                al = a_s + a_d
                al = jnp.where(al >= 0.0, al, 0.2 * al)
                ev = jnp.exp(al - cvec)
                ev = jnp.where(valid, ev, 0.0)
                gidx[...] = s16
                sidx[...] = d16
                pltpu.async_copy(hwt_hbm.at[h].at[gidx], rows, sem).wait()
                plsc.addupdate_scatter(denl, [d16], ev)
                for j in range(16):
                    ej = jnp.sum(jnp.where(iota == j, ev, 0.0))
                    for cc in range(8):
                        msg[j, pl.ds(cc * 16, 16)] = (
                            rows[j, pl.ds(cc * 16, 16)] * ej)
                pltpu.sync_copy(msg, acc.at[sidx], add=True)
                return 0

            lax.fori_loop(0, nchunk, ebody, 0)
            plsc.subcore_barrier()

            r0 = sub * RPT
            pltpu.sync_copy(
                acc.at[pl.ds(r0, RPT)],
                out_hbm.at[core].at[h].at[pl.ds(r0, RPT)])
            pltpu.sync_copy(denl, den_hbm.at[core].at[h].at[sub].at[0])
            plsc.subcore_barrier()
            return 0

        lax.fori_loop(0, heads, hbody, 0)

    return k


def _densum_body(den_ref, out_ref):
    t = pl.program_id(1)
    d = den_ref[...][:, :, 0, 0, :]

    @pl.when(t == 0)
    def _():
        out_ref[...] = d

    @pl.when(t > 0)
    def _():
        out_ref[...] += d


def _densum(denp, hh):
    denp5 = denp
    return pl.pallas_call(
        _densum_body,
        grid=(2, 16),
        in_specs=[pl.BlockSpec((1, hh, 1, 1, HALF),
                               lambda c, t: (c, 0, t, 0, 0))],
        out_specs=pl.BlockSpec((1, hh, HALF), lambda c, t: (c, 0, 0)),
        out_shape=jax.ShapeDtypeStruct((2, hh, HALF), F32),
    )(denp5)


def _mm2_body(acc_ref, den_ref, b1_ref, w2_ref, as2_ref, ad2_ref,
              hw2_ref, aso_ref, ado_ref):
    a = acc_ref[0]
    den = den_ref[0]
    out = jnp.zeros((BLK, 128), F32)
    for h in range(8):
        num = a[h]
        dh = den[h][:, None]
        o = jnp.where(dh > 0.0, num / jnp.where(dh > 0.0, dh, 1.0), 0.0)
        o = o + b1_ref[0, h, :][None, :]
        o = jnp.where(o > 0.0, o, jnp.exp(jnp.minimum(o, 0.0)) - 1.0)
        out = out + jnp.dot(o, w2_ref[h], preferred_element_type=F32)
    hw2_ref[...] = out
    aso_ref[0, 0, :] = jnp.sum(out * as2_ref[0, 0, :][None, :], axis=1)
    ado_ref[0, 0, :] = jnp.sum(out * ad2_ref[0, 0, :][None, :], axis=1)


def _mm2(acc1, den1, b1m, w2m, as2, ad2):
    return pl.pallas_call(
        _mm2_body,
        grid=(NBLK,),
        in_specs=[
            pl.BlockSpec((1, 8, BLK, 128),
                         lambda i: (i // 5, 0, i % 5, 0)),
            pl.BlockSpec((1, 8, BLK), lambda i: (i // 5, 0, i % 5)),
            pl.BlockSpec((1, 8, 128), lambda i: (0, 0, 0)),
            pl.BlockSpec((8, 128, 128), lambda i: (0, 0, 0)),
            pl.BlockSpec((1, 1, 128), lambda i: (0, 0, 0)),
            pl.BlockSpec((1, 1, 128), lambda i: (0, 0, 0)),
        ],
        out_specs=[
            pl.BlockSpec((BLK, 128), lambda i: (i, 0)),
            pl.BlockSpec((1, 1, BLK), lambda i: (0, 0, i)),
            pl.BlockSpec((1, 1, BLK), lambda i: (0, 0, i)),
        ],
        out_shape=[
            jax.ShapeDtypeStruct((NPAD, 128), F32),
            jax.ShapeDtypeStruct((1, 1, NPAD), F32),
            jax.ShapeDtypeStruct((1, 1, NPAD), F32),
        ],
    )(acc1, den1, b1m, w2m, as2, ad2)


def _fin_body(acc_ref, den_ref, b2_ref, wp1_ref, bp1_ref, wp2_ref, bp2_ref,
              zn_ref, zp_ref):
    a = acc_ref[0, 0]
    den = den_ref[0, 0][:, None]
    z = jnp.where(den > 0.0, a / jnp.where(den > 0.0, den, 1.0), 0.0)
    z = z + b2_ref[0, 0, :][None, :]
    zn_ref[...] = z
    t = jnp.dot(z, wp1_ref[...], preferred_element_type=F32)
    t = jnp.maximum(t + bp1_ref[0, 0, :][None, :], 0.0)
    t = jnp.dot(t, wp2_ref[...], preferred_element_type=F32)
    zp_ref[...] = t + bp2_ref[0, 0, :][None, :]


def _final(acc2, den2, b2m, wp1, bp1m, wp2, bp2m):
    return pl.pallas_call(
        _fin_body,
        grid=(NBLK,),
        in_specs=[
            pl.BlockSpec((1, 1, BLK, 128),
                         lambda i: (i // 5, 0, i % 5, 0)),
            pl.BlockSpec((1, 1, BLK), lambda i: (i // 5, 0, i % 5)),
            pl.BlockSpec((1, 1, 128), lambda i: (0, 0, 0)),
            pl.BlockSpec((128, 128), lambda i: (0, 0)),
            pl.BlockSpec((1, 1, 128), lambda i: (0, 0, 0)),
            pl.BlockSpec((128, 128), lambda i: (0, 0)),
            pl.BlockSpec((1, 1, 128), lambda i: (0, 0, 0)),
        ],
        out_specs=[
            pl.BlockSpec((BLK, 128), lambda i: (i, 0)),
            pl.BlockSpec((BLK, 128), lambda i: (i, 0)),
        ],
        out_shape=[
            jax.ShapeDtypeStruct((NPAD, 128), F32),
            jax.ShapeDtypeStruct((NPAD, 128), F32),
        ],
    )(acc2, den2, b2m, wp1, bp1m, wp2, bp2m)


_edge8 = _make_edge_kernel(8)
_edge1 = _make_edge_kernel(1)


def kernel(x, edge_index, community_ids, W0, b0, bn_gamma, bn_beta, W1,
           att_src1, att_dst1, b1, W2, att_src2, att_dst2, b2, Wp1, bp1,
           Wp2, bp2):
    xp = jnp.pad(x, ((0, NPAD - N), (0, 0)))
    commp = jnp.pad(community_ids, (0, NPAD - N), constant_values=127)
    comm3 = commp.reshape(NBLK, 1, BLK)
    g = (bn_gamma / jnp.sqrt(1.0 + 1e-5)).reshape(1, 1, 128)
    b0m = b0.reshape(1, 1, 128)
    betam = bn_beta.reshape(1, 1, 128)

    hf, hsum, cnt = _feat_pool(xp, comm3, W0, b0m, g, betam)

    w1a = W1[:128]
    w1b = W1[128:]
    as1 = att_src1.reshape(8, 1, 128)
    ad1 = att_dst1.reshape(8, 1, 128)
    hwt, a_s, a_d = _mm1(hf, comm3, hsum, cnt, w1a, w1b, as1, ad1)

    src = edge_index[0]
    dst = edge_index[1]
    c1 = jnp.max(a_s) + jnp.max(a_d)
    c1 = jnp.where(c1 >= 0.0, c1, 0.2 * c1)
    cvec1 = jnp.full((16,), c1, F32)
    za = jnp.zeros((RPT, 128), F32)
    zd = jnp.zeros((HALF,), F32)
    acc1, den1 = _edge8(src, dst, community_ids, hwt, a_s, a_d, cvec1,
                        za, zd)
    den1r = _densum(den1, 8)

    b1m = b1.reshape(1, 8, 128)
    w2m = W2.reshape(8, 128, 128)
    as2m = att_src2.reshape(1, 1, 128)
    ad2m = att_dst2.reshape(1, 1, 128)
    hw2, as2o, ad2o = _mm2(acc1, den1r, b1m, w2m, as2m, ad2m)

    c2 = jnp.max(as2o) + jnp.max(ad2o)
    c2 = jnp.where(c2 >= 0.0, c2, 0.2 * c2)
    cvec2 = jnp.full((16,), c2, F32)
    hw2t = hw2.reshape(1, NPAD, 128)
    acc2, den2 = _edge1(src, dst, community_ids, hw2t, as2o, ad2o, cvec2,
                        za, zd)
    den2r = _densum(den2, 1)

    b2m = b2.reshape(1, 1, 128)
    bp1m = bp1.reshape(1, 1, 128)
    bp2m = bp2.reshape(1, 1, 128)
    zn, zp = _final(acc2, den2r, b2m, Wp1, bp1m, Wp2, bp2m)
    return (zn[:N], zp[:N])
